# baseline (speedup 1.0000x reference)
import functools

import jax
import jax.numpy as jnp
from jax import lax
from jax.experimental import pallas as pl
from jax.experimental.pallas import tpu as pltpu
from jax.experimental.pallas import tpu_sc as plsc

_NC = 2
_NS = 16
_NW = _NC * _NS
_L = 16

_EPG = 32
_RPG = 3 * _EPG
_SUB = _EPG // _L
_EG_INNER = 16


def _sc_body(nvals_hbm, nodes_hbm, idx_hbm, coef_hbm, sqn_hbm, out_hbm,
             nodes_v, sqn_v, idxb0, idxb1, rows0, rows1, wbuf, coefs_v, accv,
             sem0, sem1):
    n_units = idx_hbm.shape[0] // (2 * _RPG)
    wid = lax.axis_index("s") * _NC + lax.axis_index("c")
    u_start = wid * n_units // _NW
    u_end = (wid + 1) * n_units // _NW
    g_last = 2 * u_end - 1

    pltpu.sync_copy(nodes_hbm, nodes_v)
    pltpu.sync_copy(sqn_hbm, sqn_v)
    pltpu.sync_copy(coef_hbm, coefs_v)
    cv = coefs_v[...]
    m00 = cv[0]
    m11 = cv[1]
    m22 = cv[2]
    m01 = cv[3]
    m02 = cv[4]
    m12 = cv[5]
    ws = cv[6]

    lane = lax.iota(jnp.int32, _L)
    lane3 = lane * 3

    def issue(g, idxb, rowsb, sem):
        pltpu.sync_copy(idx_hbm.at[pl.ds(g * _RPG, _RPG)], idxb)
        pltpu.async_copy(nvals_hbm.at[idxb], rowsb, sem)

    def drain(idxb, rowsb, sem):
        pltpu.make_async_copy(nvals_hbm.at[idxb], rowsb, sem).wait()

    def compute(idxb, rowsb, acc):
        for s in range(_SUB):
            n0 = plsc.load_gather(idxb, [lane3 + 48 * s])
            n1 = plsc.load_gather(idxb, [lane3 + 48 * s + 1])
            n2 = plsc.load_gather(idxb, [lane3 + 48 * s + 2])
            ids0 = n0 * 2
            ids1 = n1 * 2
            ids2 = n2 * 2
            x0 = plsc.load_gather(nodes_v, [ids0])
            y0 = plsc.load_gather(nodes_v, [ids0 + 1])
            x1 = plsc.load_gather(nodes_v, [ids1])
            y1 = plsc.load_gather(nodes_v, [ids1 + 1])
            x2 = plsc.load_gather(nodes_v, [ids2])
            y2 = plsc.load_gather(nodes_v, [ids2 + 1])
            e1x = x1 - x0
            e1y = y1 - y0
            e2x = x2 - x0
            e2y = y2 - y0
            detj = e1x * e2y - e1y * e2x
            ag = e1x * e1x + e1y * e1y
            bg = e1x * e2x + e1y * e2y
            cg = e2x * e2x + e2y * e2y
            sc = ws / detj
            sqa = plsc.load_gather(sqn_v, [n0])
            sqb = plsc.load_gather(sqn_v, [n1])
            sqc = plsc.load_gather(sqn_v, [n2])
            diag = ((detj * m00 + sc * (ag - 2.0 * bg + cg)) * sqa
                    + (detj * m11 + sc * cg) * sqb
                    + (detj * m22 + sc * ag) * sqc)
            acc = acc + diag
            wbuf[pl.ds(0, _L)] = 2.0 * (detj * m01 - sc * (cg - bg))
            wbuf[pl.ds(_L, _L)] = 2.0 * (detj * m02 - sc * (ag - bg))
            wbuf[pl.ds(2 * _L, _L)] = 2.0 * (detj * m12 - sc * bg)

            def el_body(l, a_in):
                r0 = 48 * s + l * 3
                va = rowsb[r0, pl.ds(0, _L)]
                vb = rowsb[r0 + 1, pl.ds(0, _L)]
                vc = rowsb[r0 + 2, pl.ds(0, _L)]
                d3 = va * vb
                d4 = va * vc
                d5 = vb * vc
                for j in range(_L, 256, _L):
                    va = rowsb[r0, pl.ds(j, _L)]
                    vb = rowsb[r0 + 1, pl.ds(j, _L)]
                    vc = rowsb[r0 + 2, pl.ds(j, _L)]
                    d3 = d3 + va * vb
                    d4 = d4 + va * vc
                    d5 = d5 + vb * vc
                li = jnp.full((_L,), l, dtype=jnp.int32)
                w3 = plsc.load_gather(wbuf, [li])
                w4 = plsc.load_gather(wbuf, [li + _L])
                w5 = plsc.load_gather(wbuf, [li + 2 * _L])
                return a_in + (w3 * d3 + w4 * d4) + w5 * d5

            acc = plsc.parallel_loop(0, _EG_INNER, unroll=2, carry=acc)(el_body)
        return acc

    issue(2 * u_start, idxb0, rows0, sem0)

    def unit_body(u, acc):
        issue(2 * u + 1, idxb1, rows1, sem1)
        drain(idxb0, rows0, sem0)
        acc = compute(idxb0, rows0, acc)
        issue(jnp.minimum(2 * u + 2, g_last), idxb0, rows0, sem0)
        drain(idxb1, rows1, sem1)
        acc = compute(idxb1, rows1, acc)
        return acc

    acc = lax.fori_loop(u_start, u_end, unit_body,
                        jnp.zeros((_L,), jnp.float32))
    drain(idxb0, rows0, sem0)
    accv[...] = acc
    pltpu.sync_copy(accv, out_hbm.at[wid])


def _sqnorm_tc_body(v_ref, o_ref):
    v = v_ref[...]
    o_ref[...] = jnp.sum(v * v, axis=1)


def _row_sqnorms(nvals):
    n, d = nvals.shape
    return pl.pallas_call(
        _sqnorm_tc_body,
        out_shape=jax.ShapeDtypeStruct((n,), jnp.float32),
    )(nvals)


def _run_sc(nvals, nodes_xy, idx_flat, coefs, sqn):
    n_nodes = nodes_xy.shape[0] // 2
    mesh = plsc.VectorSubcoreMesh(core_axis_name="c", subcore_axis_name="s")
    f = pl.kernel(
        _sc_body,
        out_type=jax.ShapeDtypeStruct((_NW, _L), jnp.float32),
        mesh=mesh,
        scratch_types=[
            pltpu.VMEM((2 * n_nodes,), jnp.float32),
            pltpu.VMEM((n_nodes,), jnp.float32),
            pltpu.VMEM((_RPG,), jnp.int32),
            pltpu.VMEM((_RPG,), jnp.int32),
            pltpu.VMEM((_RPG, 256), jnp.float32),
            pltpu.VMEM((_RPG, 256), jnp.float32),
            pltpu.VMEM((3 * _L,), jnp.float32),
            pltpu.VMEM((_L,), jnp.float32),
            pltpu.VMEM((_L,), jnp.float32),
            pltpu.SemaphoreType.DMA,
            pltpu.SemaphoreType.DMA,
        ],
        compiler_params=pltpu.CompilerParams(needs_layout_passes=False),
    )
    return f(nvals, nodes_xy, idx_flat, coefs, sqn)


def kernel(nodal_values, nodes, elements, quad_points, quad_weights):
    qx = quad_points[:, 0]
    qy = quad_points[:, 1]
    shp = jnp.stack([1.0 - qx - qy, qx, qy], axis=1)
    m = jnp.einsum('q,qa,qb->ab', quad_weights, shp, shp)
    coefs = jnp.zeros((16,), jnp.float32)
    coefs = coefs.at[:7].set(jnp.stack(
        [m[0, 0], m[1, 1], m[2, 2], m[0, 1], m[0, 2], m[1, 2],
         jnp.sum(quad_weights)]))
    idx_flat = elements.reshape(-1).astype(jnp.int32)
    sqn = _row_sqnorms(nodal_values)
    partials = _run_sc(nodal_values, nodes.reshape(-1), idx_flat, coefs, sqn)
    return jnp.sum(partials)


# speedup vs baseline: 1.5876x; 1.5876x over previous
import functools

import jax
import jax.numpy as jnp
from jax import lax
from jax.experimental import pallas as pl
from jax.experimental.pallas import tpu as pltpu
from jax.experimental.pallas import tpu_sc as plsc

_NC = 2
_NS = 16
_NW = _NC * _NS
_L = 16

_EPG = 32
_RPG = 3 * _EPG
_SUB = _EPG // _L
_EG_INNER = 16


def _sc_body(nvals_hbm, nodes_hbm, idx_hbm, coef_hbm, sqn_hbm, out_hbm,
             nodes_v, sqn_v, idxb0, idxb1, rows0, rows1, wbuf, coefs_v, accv,
             sem0, sem1):
    n_units = idx_hbm.shape[0] // (2 * _RPG)
    wid = lax.axis_index("s") * _NC + lax.axis_index("c")
    u_start = wid * n_units // _NW
    u_end = (wid + 1) * n_units // _NW
    g_last = 2 * u_end - 1

    pltpu.sync_copy(nodes_hbm, nodes_v)
    pltpu.sync_copy(sqn_hbm, sqn_v)
    pltpu.sync_copy(coef_hbm, coefs_v)
    cv = coefs_v[...]
    m00 = cv[0]
    m11 = cv[1]
    m22 = cv[2]
    m01 = cv[3]
    m02 = cv[4]
    m12 = cv[5]
    ws = cv[6]

    lane = lax.iota(jnp.int32, _L)
    lane3 = lane * 3

    def issue(g, idxb, rowsb, sem):
        pltpu.sync_copy(idx_hbm.at[pl.ds(g * _RPG, _RPG)], idxb)
        pltpu.async_copy(nvals_hbm.at[idxb], rowsb, sem)

    def drain(idxb, rowsb, sem):
        pltpu.make_async_copy(nvals_hbm.at[idxb], rowsb, sem).wait()

    def compute(idxb, rowsb, acc):
        for s in range(_SUB):
            n0 = plsc.load_gather(idxb, [lane3 + 48 * s])
            n1 = plsc.load_gather(idxb, [lane3 + 48 * s + 1])
            n2 = plsc.load_gather(idxb, [lane3 + 48 * s + 2])
            ids0 = n0 * 2
            ids1 = n1 * 2
            ids2 = n2 * 2
            x0 = plsc.load_gather(nodes_v, [ids0])
            y0 = plsc.load_gather(nodes_v, [ids0 + 1])
            x1 = plsc.load_gather(nodes_v, [ids1])
            y1 = plsc.load_gather(nodes_v, [ids1 + 1])
            x2 = plsc.load_gather(nodes_v, [ids2])
            y2 = plsc.load_gather(nodes_v, [ids2 + 1])
            e1x = x1 - x0
            e1y = y1 - y0
            e2x = x2 - x0
            e2y = y2 - y0
            detj = e1x * e2y - e1y * e2x
            ag = e1x * e1x + e1y * e1y
            bg = e1x * e2x + e1y * e2y
            cg = e2x * e2x + e2y * e2y
            sc = ws / detj
            sqa = plsc.load_gather(sqn_v, [n0])
            sqb = plsc.load_gather(sqn_v, [n1])
            sqc = plsc.load_gather(sqn_v, [n2])
            diag = ((detj * m00 + sc * (ag - 2.0 * bg + cg)) * sqa
                    + (detj * m11 + sc * cg) * sqb
                    + (detj * m22 + sc * ag) * sqc)
            acc = acc + diag
            wbuf[pl.ds(0, _L)] = 2.0 * (detj * m01 - sc * (cg - bg))
            wbuf[pl.ds(_L, _L)] = 2.0 * (detj * m02 - sc * (ag - bg))
            wbuf[pl.ds(2 * _L, _L)] = 2.0 * (detj * m12 - sc * bg)

            def el_body(l, a_in):
                r0 = 48 * s + l * 3
                bf = jnp.bfloat16
                va = plsc.bitcast(rowsb[r0, pl.ds(0, _L)], bf)
                vb = plsc.bitcast(rowsb[r0 + 1, pl.ds(0, _L)], bf)
                vc = plsc.bitcast(rowsb[r0 + 2, pl.ds(0, _L)], bf)
                d3 = va * vb
                d4 = va * vc
                d5 = vb * vc
                for j in range(_L, 128, _L):
                    va = plsc.bitcast(rowsb[r0, pl.ds(j, _L)], bf)
                    vb = plsc.bitcast(rowsb[r0 + 1, pl.ds(j, _L)], bf)
                    vc = plsc.bitcast(rowsb[r0 + 2, pl.ds(j, _L)], bf)
                    d3 = d3 + va * vb
                    d4 = d4 + va * vc
                    d5 = d5 + vb * vc
                d3a, d3b = plsc.unpack(d3, format=plsc.PackFormat.INTERLEAVED)
                d4a, d4b = plsc.unpack(d4, format=plsc.PackFormat.INTERLEAVED)
                d5a, d5b = plsc.unpack(d5, format=plsc.PackFormat.INTERLEAVED)
                li = jnp.full((_L,), l, dtype=jnp.int32)
                w3 = plsc.load_gather(wbuf, [li])
                w4 = plsc.load_gather(wbuf, [li + _L])
                w5 = plsc.load_gather(wbuf, [li + 2 * _L])
                return (a_in + (w3 * (d3a + d3b) + w4 * (d4a + d4b))
                        + w5 * (d5a + d5b))

            acc = plsc.parallel_loop(0, _EG_INNER, unroll=2, carry=acc)(el_body)
        return acc

    issue(2 * u_start, idxb0, rows0, sem0)

    def unit_body(u, acc):
        issue(2 * u + 1, idxb1, rows1, sem1)
        drain(idxb0, rows0, sem0)
        acc = compute(idxb0, rows0, acc)
        issue(jnp.minimum(2 * u + 2, g_last), idxb0, rows0, sem0)
        drain(idxb1, rows1, sem1)
        acc = compute(idxb1, rows1, acc)
        return acc

    acc = lax.fori_loop(u_start, u_end, unit_body,
                        jnp.zeros((_L,), jnp.float32))
    drain(idxb0, rows0, sem0)
    accv[...] = acc
    pltpu.sync_copy(accv, out_hbm.at[wid])


def _sqnorm_tc_body(v_ref, o_ref):
    v = v_ref[...]
    o_ref[...] = jnp.sum(v * v, axis=1)


def _row_sqnorms(nvals):
    n, d = nvals.shape
    return pl.pallas_call(
        _sqnorm_tc_body,
        out_shape=jax.ShapeDtypeStruct((n,), jnp.float32),
    )(nvals)


def _run_sc(nvals, nodes_xy, idx_flat, coefs, sqn):
    n_nodes = nodes_xy.shape[0] // 2
    mesh = plsc.VectorSubcoreMesh(core_axis_name="c", subcore_axis_name="s")
    f = pl.kernel(
        _sc_body,
        out_type=jax.ShapeDtypeStruct((_NW, _L), jnp.float32),
        mesh=mesh,
        scratch_types=[
            pltpu.VMEM((2 * n_nodes,), jnp.float32),
            pltpu.VMEM((n_nodes,), jnp.float32),
            pltpu.VMEM((_RPG,), jnp.int32),
            pltpu.VMEM((_RPG,), jnp.int32),
            pltpu.VMEM((_RPG, 128), jnp.int32),
            pltpu.VMEM((_RPG, 128), jnp.int32),
            pltpu.VMEM((3 * _L,), jnp.float32),
            pltpu.VMEM((_L,), jnp.float32),
            pltpu.VMEM((_L,), jnp.float32),
            pltpu.SemaphoreType.DMA,
            pltpu.SemaphoreType.DMA,
        ],
        compiler_params=pltpu.CompilerParams(needs_layout_passes=False),
    )
    return f(nvals, nodes_xy, idx_flat, coefs, sqn)


def kernel(nodal_values, nodes, elements, quad_points, quad_weights):
    qx = quad_points[:, 0]
    qy = quad_points[:, 1]
    shp = jnp.stack([1.0 - qx - qy, qx, qy], axis=1)
    m = jnp.einsum('q,qa,qb->ab', quad_weights, shp, shp)
    coefs = jnp.zeros((16,), jnp.float32)
    coefs = coefs.at[:7].set(jnp.stack(
        [m[0, 0], m[1, 1], m[2, 2], m[0, 1], m[0, 2], m[1, 2],
         jnp.sum(quad_weights)]))
    idx_flat = elements.reshape(-1).astype(jnp.int32)
    sqn = _row_sqnorms(nodal_values)
    n, d = nodal_values.shape
    vpacked = jax.lax.bitcast_convert_type(
        nodal_values.astype(jnp.bfloat16).reshape(n, d // 2, 2), jnp.int32)
    partials = _run_sc(vpacked, nodes.reshape(-1), idx_flat, coefs, sqn)
    return jnp.sum(partials)
